# baseline (speedup 1.0000x reference)
import functools

import jax
import jax.numpy as jnp
from jax import lax
from jax.experimental import pallas as pl
from jax.experimental.pallas import tpu as pltpu
from jax.experimental.pallas import tpu_sc as plsc

BF, S, P = 1024, 200, 64
NIDX = BF * S

NC, NS, L = 2, 16, 16
NW = NC * NS
IDX_W = NIDX // NW

BLOCK_ROWS = 128


@functools.partial(
    pl.kernel,
    out_type=jax.ShapeDtypeStruct((NIDX,), jnp.float32),
    mesh=plsc.VectorSubcoreMesh(core_axis_name="c", subcore_axis_name="s"),
    compiler_params=pltpu.CompilerParams(needs_layout_passes=False),
    scratch_types=[
        pltpu.VMEM((1024,), jnp.float32),
        pltpu.VMEM((IDX_W,), jnp.int32),
        pltpu.VMEM((IDX_W,), jnp.float32),
        pltpu.SemaphoreType.DMA,
        pltpu.SemaphoreType.DMA,
    ],
)
def _sc_gather(gamma_hbm, t_hbm, g_hbm, tbl_v, idx_v, val_v, sem_t, sem_i):
    wid = lax.axis_index("s") * NC + lax.axis_index("c")
    base = wid * IDX_W
    ct = pltpu.async_copy(gamma_hbm, tbl_v.at[pl.ds(0, 1000)], sem_t)
    ci = pltpu.async_copy(t_hbm.at[pl.ds(base, IDX_W)], idx_v, sem_i)
    ct.wait()
    ci.wait()

    @plsc.parallel_loop(0, IDX_W, step=L, unroll=8)
    def _gather_body(off):
        iv = idx_v[pl.ds(off, L)]
        val_v[pl.ds(off, L)] = plsc.load_gather(tbl_v, [iv])

    pltpu.sync_copy(val_v, g_hbm.at[pl.ds(base, IDX_W)])


def _dense_body(g_ref, x_ref, n_ref, o_ref, e_ref):
    @pl.when(pl.program_id(0) == 0)
    def _build_e():
        col_s = lax.broadcasted_iota(jnp.int32, (S, S * P), 1) // P
        row_s = lax.broadcasted_iota(jnp.int32, (S, S * P), 0)
        e_ref[...] = (col_s == row_s).astype(jnp.bfloat16)

    gb = g_ref[...]
    a = jnp.sqrt(gb).astype(jnp.bfloat16)
    b = jnp.sqrt(1.0 - gb).astype(jnp.bfloat16)
    e = e_ref[...]
    a2 = jnp.dot(a, e, preferred_element_type=jnp.float32)
    b2 = jnp.dot(b, e, preferred_element_type=jnp.float32)
    o_ref[...] = a2 * x_ref[...] + b2 * n_ref[...]


def _dense(g, x2, n2):
    grid = (BF // BLOCK_ROWS,)
    return pl.pallas_call(
        _dense_body,
        grid=grid,
        in_specs=[
            pl.BlockSpec((BLOCK_ROWS, S), lambda i: (i, 0)),
            pl.BlockSpec((BLOCK_ROWS, S * P), lambda i: (i, 0)),
            pl.BlockSpec((BLOCK_ROWS, S * P), lambda i: (i, 0)),
        ],
        out_specs=pl.BlockSpec((BLOCK_ROWS, S * P), lambda i: (i, 0)),
        out_shape=jax.ShapeDtypeStruct((BF, S * P), jnp.float32),
        scratch_shapes=[pltpu.VMEM((S, S * P), jnp.bfloat16)],
    )(g, x2, n2)


def kernel(x, gamma, noise, t):
    g = _sc_gather(gamma, t.reshape(NIDX))
    out = _dense(g.reshape(BF, S), x.reshape(BF, S * P), noise.reshape(BF, S * P))
    return (out.reshape(BF, S, P), noise, t)


# speedup vs baseline: 1.0046x; 1.0046x over previous
import functools

import jax
import jax.numpy as jnp
from jax import lax
from jax.experimental import pallas as pl
from jax.experimental.pallas import tpu as pltpu
from jax.experimental.pallas import tpu_sc as plsc

BF, S, P = 1024, 200, 64
NIDX = BF * S

NC, NS, L = 2, 16, 16
NW = NC * NS
IDX_W = NIDX // NW

BLOCK_ROWS = 128


@functools.partial(
    pl.kernel,
    out_type=jax.ShapeDtypeStruct((NIDX,), jnp.float32),
    mesh=plsc.VectorSubcoreMesh(core_axis_name="c", subcore_axis_name="s"),
    compiler_params=pltpu.CompilerParams(needs_layout_passes=False),
    scratch_types=[
        pltpu.VMEM((1024,), jnp.float32),
        pltpu.VMEM((IDX_W,), jnp.int32),
        pltpu.VMEM((IDX_W,), jnp.float32),
    ],
)
def _sc_gather(gamma_hbm, t_hbm, g_hbm, tbl_v, idx_v, val_v):
    wid = lax.axis_index("s") * NC + lax.axis_index("c")
    base = wid * IDX_W
    pltpu.sync_copy(gamma_hbm, tbl_v.at[pl.ds(0, 1000)])
    pltpu.sync_copy(t_hbm.at[pl.ds(base, IDX_W)], idx_v)

    def _gather_body(j, carry):
        off = j * L
        iv = idx_v[pl.ds(off, L)]
        val_v[pl.ds(off, L)] = plsc.load_gather(tbl_v, [iv])
        return carry

    lax.fori_loop(0, IDX_W // L, _gather_body, 0)

    pltpu.sync_copy(val_v, g_hbm.at[pl.ds(base, IDX_W)])


def _dense_body(g_ref, x_ref, n_ref, o_ref, e_ref):
    @pl.when(pl.program_id(0) == 0)
    def _build_e():
        col_s = lax.broadcasted_iota(jnp.int32, (S, S * P), 1) // P
        row_s = lax.broadcasted_iota(jnp.int32, (S, S * P), 0)
        e_ref[...] = (col_s == row_s).astype(jnp.bfloat16)

    gb = g_ref[...]
    a = jnp.sqrt(gb).astype(jnp.bfloat16)
    b = jnp.sqrt(1.0 - gb).astype(jnp.bfloat16)
    e = e_ref[...]
    a2 = jnp.dot(a, e, preferred_element_type=jnp.float32)
    b2 = jnp.dot(b, e, preferred_element_type=jnp.float32)
    o_ref[...] = a2 * x_ref[...] + b2 * n_ref[...]


def _dense(g, x2, n2):
    grid = (BF // BLOCK_ROWS,)
    return pl.pallas_call(
        _dense_body,
        grid=grid,
        in_specs=[
            pl.BlockSpec((BLOCK_ROWS, S), lambda i: (i, 0)),
            pl.BlockSpec((BLOCK_ROWS, S * P), lambda i: (i, 0)),
            pl.BlockSpec((BLOCK_ROWS, S * P), lambda i: (i, 0)),
        ],
        out_specs=pl.BlockSpec((BLOCK_ROWS, S * P), lambda i: (i, 0)),
        out_shape=jax.ShapeDtypeStruct((BF, S * P), jnp.float32),
        scratch_shapes=[pltpu.VMEM((S, S * P), jnp.bfloat16)],
    )(g, x2, n2)


def kernel(x, gamma, noise, t):
    g = _sc_gather(gamma, t.reshape(NIDX))
    out = _dense(g.reshape(BF, S), x.reshape(BF, S * P), noise.reshape(BF, S * P))
    return (out.reshape(BF, S, P), noise, t)
